# Initial kernel scaffold; baseline (speedup 1.0000x reference)
import functools

import jax
import jax.numpy as jnp
from jax import lax
from jax.experimental import pallas as pl
from jax.experimental.pallas import tpu as pltpu
from jax.experimental.pallas import tpu_sc as plsc

N = 10000
E = 320000
D = 128
V = 100000
S = 1024

NC = 2
NS = 16
NW = NC * NS

ROWS_PER_W = 320
GCH = 80

EPW = E // NW
G = 128
AGG_ROWS = 1040
RPT = AGG_ROWS // NS
TRASH = S


def _emb_body(nid_hbm, table_hbm, out_hbm, idx_v, rows_v, sem):
    c = lax.axis_index("c")
    s = lax.axis_index("s")
    w = s * NC + c
    base = w * ROWS_PER_W
    for k in range(ROWS_PER_W // GCH):
        b = base + k * GCH

        @pl.when(b < N)
        def _():
            pltpu.sync_copy(nid_hbm.at[pl.ds(b, GCH)], idx_v)
            pltpu.async_copy(table_hbm.at[idx_v], rows_v, sem).wait()
            pltpu.sync_copy(rows_v, out_hbm.at[pl.ds(b, GCH)])


def _edge_body(src_hbm, dst_hbm, h_hbm, z128_hbm, z16_hbm, ones_hbm,
               agg_out, deg_out,
               esrc, edst, sel_src, sel_dst, stage_src, stage_dst,
               rows_v, ones_v, agg_sh, deg_sh, sem):
    c = lax.axis_index("c")
    s = lax.axis_index("s")
    w = s * NC + c

    pltpu.sync_copy(src_hbm.at[pl.ds(w * EPW, EPW)], esrc)
    pltpu.sync_copy(dst_hbm.at[pl.ds(w * EPW, EPW)], edst)
    pltpu.sync_copy(z128_hbm.at[pl.ds(s * RPT, RPT)], agg_sh.at[pl.ds(s * RPT, RPT)])
    pltpu.sync_copy(z16_hbm.at[pl.ds(s * RPT, RPT)], deg_sh.at[pl.ds(s * RPT, RPT)])
    pltpu.sync_copy(ones_hbm, ones_v)
    plsc.subcore_barrier()

    def cbody(i, cnt):
        dv = edst[pl.ds(i * 16, 16)]
        sv = esrc[pl.ds(i * 16, 16)]
        m = dv < jnp.int32(S)
        plsc.store_compressed(sel_dst.at[pl.ds(cnt, 16)], dv, mask=m)
        plsc.store_compressed(sel_src.at[pl.ds(cnt, 16)], sv, mask=m)
        return cnt + jnp.max(plsc.all_reduce_population_count(m))

    cnt = lax.fori_loop(0, EPW // 16, cbody, jnp.int32(0))

    for k in range(G // 16):
        sel_dst[pl.ds(cnt + 16 * k, 16)] = jnp.full((16,), TRASH, jnp.int32)
        sel_src[pl.ds(cnt + 16 * k, 16)] = jnp.zeros((16,), jnp.int32)

    def gbody(ci, carry):
        base = ci * G
        for k in range(G // 16):
            stage_src[pl.ds(16 * k, 16)] = sel_src[pl.ds(base + 16 * k, 16)]
            stage_dst[pl.ds(16 * k, 16)] = sel_dst[pl.ds(base + 16 * k, 16)]
        pltpu.async_copy(h_hbm.at[stage_src], rows_v, sem).wait()
        pltpu.sync_copy(rows_v, agg_sh.at[stage_dst], add=True)
        pltpu.sync_copy(ones_v, deg_sh.at[stage_dst], add=True)
        return carry

    nch = (cnt + (G - 1)) >> 7
    lax.fori_loop(0, nch, gbody, jnp.int32(0))
    plsc.subcore_barrier()

    pltpu.sync_copy(agg_sh.at[pl.ds(s * RPT, RPT)], agg_out.at[c, pl.ds(s * RPT, RPT)])
    pltpu.sync_copy(deg_sh.at[pl.ds(s * RPT, RPT)], deg_out.at[c, pl.ds(s * RPT, RPT)])


def _enc_body(x_ref, emb_ref, w_ref, b_ref, o_ref):
    o_ref[...] = (
        jnp.dot(x_ref[...], w_ref[...], preferred_element_type=jnp.float32)
        + b_ref[...]
        + emb_ref[...]
    )


def _final_body(h_ref, agg_ref, deg_ref, ws_ref, wn_ref, bg_ref, wm_ref,
                bm_ref, o_ref):
    a = agg_ref[...]
    agg = a[0, :S, :] + a[1, :S, :]
    dg = deg_ref[...]
    deg = dg[0, :S, 0:1] + dg[1, :S, 0:1]
    mean = agg * (1.0 / jnp.maximum(deg, 1.0))
    h2 = jnp.maximum(
        jnp.dot(h_ref[...], ws_ref[...], preferred_element_type=jnp.float32)
        + jnp.dot(mean, wn_ref[...], preferred_element_type=jnp.float32)
        + bg_ref[...],
        0.0,
    )
    o_ref[...] = jnp.dot(h2, wm_ref[...], preferred_element_type=jnp.float32) + bm_ref[...]


def kernel(x, edge_index, n_id, seed_count, W_enc, b_enc, emb_table,
           W_self, W_neigh, b_gnn, W_mlp, b_mlp):
    del seed_count
    mesh = plsc.VectorSubcoreMesh(core_axis_name="c", subcore_axis_name="s")

    emb_gather = functools.partial(
        pl.kernel,
        out_type=jax.ShapeDtypeStruct((N, D), jnp.float32),
        mesh=mesh,
        scratch_types=[
            pltpu.VMEM((GCH,), jnp.int32),
            pltpu.VMEM((GCH, D), jnp.float32),
            pltpu.SemaphoreType.DMA,
        ],
    )(_emb_body)
    emb_rows = emb_gather(n_id, emb_table)

    R = 1250
    h = pl.pallas_call(
        _enc_body,
        grid=(N // R,),
        in_specs=[
            pl.BlockSpec((R, D), lambda i: (i, 0)),
            pl.BlockSpec((R, D), lambda i: (i, 0)),
            pl.BlockSpec((D, D), lambda i: (0, 0)),
            pl.BlockSpec((1, D), lambda i: (0, 0)),
        ],
        out_specs=pl.BlockSpec((R, D), lambda i: (i, 0)),
        out_shape=jax.ShapeDtypeStruct((N, D), jnp.float32),
    )(x, emb_rows, W_enc, b_enc.reshape(1, D))

    src = edge_index[0]
    dst = edge_index[1]
    z128 = jnp.zeros((AGG_ROWS, D), jnp.float32)
    z16 = jnp.zeros((AGG_ROWS, 16), jnp.float32)
    ones16 = jnp.ones((G, 16), jnp.float32)
    edge_agg = functools.partial(
        pl.kernel,
        out_type=(
            jax.ShapeDtypeStruct((NC, AGG_ROWS, D), jnp.float32),
            jax.ShapeDtypeStruct((NC, AGG_ROWS, 16), jnp.float32),
        ),
        mesh=mesh,
        scratch_types=[
            pltpu.VMEM((EPW,), jnp.int32),
            pltpu.VMEM((EPW,), jnp.int32),
            pltpu.VMEM((EPW + G,), jnp.int32),
            pltpu.VMEM((EPW + G,), jnp.int32),
            pltpu.VMEM((G,), jnp.int32),
            pltpu.VMEM((G,), jnp.int32),
            pltpu.VMEM((G, D), jnp.float32),
            pltpu.VMEM((G, 16), jnp.float32),
            pltpu.VMEM_SHARED((AGG_ROWS, D), jnp.float32),
            pltpu.VMEM_SHARED((AGG_ROWS, 16), jnp.float32),
            pltpu.SemaphoreType.DMA,
        ],
    )(_edge_body)
    agg2, deg2 = edge_agg(src, dst, h, z128, z16, ones16)

    out = pl.pallas_call(
        _final_body,
        out_shape=jax.ShapeDtypeStruct((S, 1), jnp.float32),
    )(h[:S], agg2, deg2, W_self, W_neigh, b_gnn.reshape(1, D),
      W_mlp, b_mlp.reshape(1, 1))
    return jnp.squeeze(out, axis=-1)


# speedup vs baseline: 6.9142x; 6.9142x over previous
import functools

import jax
import jax.numpy as jnp
from jax import lax
from jax.experimental import pallas as pl
from jax.experimental.pallas import tpu as pltpu
from jax.experimental.pallas import tpu_sc as plsc

N = 10000
E = 320000
D = 128
V = 100000
S = 1024

NC = 2
NS = 16
NW = NC * NS

ROWS_PER_W = 320
GCH = 80

EPW = E // NW
G = 64
REG = 2080
AGG_ROWS = 1152
RPT = AGG_ROWS // NS
TRASH = S


def _emb_body(nid_hbm, table_hbm, out_hbm, idx_v, rows_v, sem):
    c = lax.axis_index("c")
    s = lax.axis_index("s")
    w = s * NC + c
    base = w * ROWS_PER_W
    for k in range(ROWS_PER_W // GCH):
        b = base + k * GCH

        @pl.when(b < N)
        def _():
            pltpu.sync_copy(nid_hbm.at[pl.ds(b, GCH)], idx_v)
            pltpu.async_copy(table_hbm.at[idx_v], rows_v, sem).wait()
            pltpu.sync_copy(rows_v, out_hbm.at[pl.ds(b, GCH)])


def _edge_body(src_hbm, dst_hbm, h_hbm, z128_hbm, ones_hbm,
               acc_out,
               esrc, edst, sel_src, sel_dst, chunk_base,
               ss0, ss1, ss2, ss3, st0, st1, st2, st3,
               rw0, rw1, rw2, rw3, acc_sh,
               g0, g1, g2, g3, sc0, sc1, sc2, sc3):
    c = lax.axis_index("c")
    s = lax.axis_index("s")
    w = s * NC + c

    pltpu.sync_copy(src_hbm.at[pl.ds(w * EPW, EPW)], esrc)
    pltpu.sync_copy(dst_hbm.at[pl.ds(w * EPW, EPW)], edst)
    pltpu.sync_copy(z128_hbm.at[pl.ds(0, 2 * RPT)],
                    acc_sh.at[pl.ds(s * 2 * RPT, 2 * RPT)])
    SS = [ss0, ss1, ss2, ss3]
    ST = [st0, st1, st2, st3]
    RW = [rw0, rw1, rw2, rw3]
    GS = [g0, g1, g2, g3]
    SC = [sc0, sc1, sc2, sc3]
    for b in range(4):
        pltpu.sync_copy(ones_hbm, RW[b].at[pl.ds(G, G)])
    plsc.subcore_barrier()

    lane = jax.lax.iota(jnp.int32, 16)
    shidx = [jnp.maximum(lane - k, 0) for k in (1, 2, 4, 8)]
    shkeep = [jnp.int32(1) - lax.shift_right_logical(lane - k, 31)
              for k in (1, 2, 4, 8)]
    target = lane + jnp.int32(1)
    pos0 = jnp.zeros((16,), jnp.int32)

    def cbody(i, cnts):
        out = []
        for u in range(5):
            cnt = cnts[u]
            off = i * 80 + u * 16
            dv = edst[pl.ds(off, 16)]
            sv = esrc[pl.ds(off, 16)]
            mi = lax.shift_right_logical(dv - jnp.int32(S), 31)
            ips = mi
            for k in range(4):
                sh = ips.at[shidx[k]].get(mode="promise_in_bounds")
                ips = ips + sh * shkeep[k]
            pos = pos0
            for step in (8, 4, 2, 1):
                probe = ips.at[pos + jnp.int32(step - 1)].get(mode="promise_in_bounds")
                ok = lax.shift_right_logical(probe - target, 31)
                pos = pos + ok * jnp.int32(step)
            g = jnp.minimum(pos, jnp.int32(15))
            sel_dst[pl.ds(u * REG + cnt, 16)] = dv.at[g].get(mode="promise_in_bounds")
            sel_src[pl.ds(u * REG + cnt, 16)] = sv.at[g].get(mode="promise_in_bounds")
            out.append(cnt + ips[15])
        return tuple(out)

    cnts = lax.fori_loop(0, EPW // 80, cbody, (jnp.int32(0),) * 5)

    for u in range(5):
        for k in range(G // 16):
            sel_dst[pl.ds(u * REG + cnts[u] + 16 * k, 16)] = jnp.full((16,), TRASH, jnp.int32)
            sel_src[pl.ds(u * REG + cnts[u] + 16 * k, 16)] = jnp.zeros((16,), jnp.int32)

    def bbody(u):
        def body(j, idx):
            chunk_base[pl.ds(idx * 16, 16)] = (
                jnp.full((16,), u * REG, jnp.int32) + j * jnp.int32(G))
            return idx + jnp.int32(1)
        return body

    nch = jnp.int32(0)
    for u in range(5):
        nch = lax.fori_loop(0, (cnts[u] + (G - 1)) >> 6, bbody(u), nch)

    def fill_stage(j, b):
        base = chunk_base[pl.ds(j * 16, 16)][0]
        for k in range(G // 16):
            dvk = sel_dst[pl.ds(base + 16 * k, 16)]
            ST[b][pl.ds(16 * k, 16)] = dvk
            ST[b][pl.ds(G + 16 * k, 16)] = dvk + jnp.int32(AGG_ROWS)
            SS[b][pl.ds(16 * k, 16)] = sel_src[pl.ds(base + 16 * k, 16)]

    def gather_start(b):
        pltpu.async_copy(h_hbm.at[SS[b]], RW[b].at[pl.ds(0, G)], GS[b])

    def gather_wait(b):
        pltpu.make_async_copy(h_hbm.at[SS[b]], RW[b].at[pl.ds(0, G)],
                              GS[b]).wait()

    def scatter_start(b):
        pltpu.async_copy(RW[b], acc_sh.at[ST[b]], SC[b], add=True)

    def scatter_wait(b):
        pltpu.make_async_copy(RW[b], acc_sh.at[ST[b]], SC[b]).wait()

    for k in range(3):
        @pl.when(k < nch)
        def _(k=k):
            fill_stage(jnp.int32(k), k)
            gather_start(k)

    def quad_body(q, carry):
        j0 = q * 4
        for b in range(4):
            j = j0 + b

            @pl.when(j < nch)
            def _(b=b, j=j):
                gather_wait(b)
                scatter_start(b)

                @pl.when(j + 3 < nch)
                def _(b=b, j=j):
                    nb = (b + 3) % 4

                    @pl.when(j >= 1)
                    def _(nb=nb):
                        scatter_wait(nb)

                    fill_stage(j + 3, nb)
                    gather_start(nb)

        return carry

    lax.fori_loop(0, (nch + 3) >> 2, quad_body, jnp.int32(0))

    for b in range(4):
        @pl.when(b < nch)
        def _(b=b):
            scatter_wait(b)

    plsc.subcore_barrier()

    pltpu.sync_copy(acc_sh.at[pl.ds(s * 2 * RPT, 2 * RPT)],
                    acc_out.at[c, pl.ds(s * 2 * RPT, 2 * RPT)])


def _enc_body(x_ref, emb_ref, w_ref, b_ref, o_ref):
    o_ref[...] = (
        jnp.dot(x_ref[...], w_ref[...], preferred_element_type=jnp.float32)
        + b_ref[...]
        + emb_ref[...]
    )


def _final_body(h_ref, acc_ref, ws_ref, wn_ref, bg_ref, wm_ref,
                bm_ref, o_ref):
    a = acc_ref[...]
    agg = a[0, :S, :] + a[1, :S, :]
    deg = a[0, AGG_ROWS:AGG_ROWS + S, 0:1] + a[1, AGG_ROWS:AGG_ROWS + S, 0:1]
    mean = agg * (1.0 / jnp.maximum(deg, 1.0))
    h2 = jnp.maximum(
        jnp.dot(h_ref[...], ws_ref[...], preferred_element_type=jnp.float32)
        + jnp.dot(mean, wn_ref[...], preferred_element_type=jnp.float32)
        + bg_ref[...],
        0.0,
    )
    o_ref[...] = jnp.dot(h2, wm_ref[...], preferred_element_type=jnp.float32) + bm_ref[...]


def kernel(x, edge_index, n_id, seed_count, W_enc, b_enc, emb_table,
           W_self, W_neigh, b_gnn, W_mlp, b_mlp):
    del seed_count
    mesh = plsc.VectorSubcoreMesh(core_axis_name="c", subcore_axis_name="s")

    emb_gather = functools.partial(
        pl.kernel,
        out_type=jax.ShapeDtypeStruct((N, D), jnp.float32),
        mesh=mesh,
        scratch_types=[
            pltpu.VMEM((GCH,), jnp.int32),
            pltpu.VMEM((GCH, D), jnp.float32),
            pltpu.SemaphoreType.DMA,
        ],
    )(_emb_body)
    emb_rows = emb_gather(n_id, emb_table)

    R = 2000
    h = pl.pallas_call(
        _enc_body,
        grid=(N // R,),
        in_specs=[
            pl.BlockSpec((R, D), lambda i: (i, 0)),
            pl.BlockSpec((R, D), lambda i: (i, 0)),
            pl.BlockSpec((D, D), lambda i: (0, 0)),
            pl.BlockSpec((1, D), lambda i: (0, 0)),
        ],
        out_specs=pl.BlockSpec((R, D), lambda i: (i, 0)),
        out_shape=jax.ShapeDtypeStruct((N, D), jnp.float32),
    )(x, emb_rows, W_enc, b_enc.reshape(1, D))

    src = edge_index[0]
    dst = edge_index[1]
    z128 = jnp.zeros((2 * RPT, D), jnp.float32)
    ones128 = jnp.ones((G, D), jnp.float32)
    edge_agg = functools.partial(
        pl.kernel,
        out_type=jax.ShapeDtypeStruct((NC, 2 * AGG_ROWS, D), jnp.float32),
        mesh=mesh,
        scratch_types=[
            pltpu.VMEM((EPW,), jnp.int32),
            pltpu.VMEM((EPW,), jnp.int32),
            pltpu.VMEM((5 * REG,), jnp.int32),
            pltpu.VMEM((5 * REG,), jnp.int32),
            pltpu.VMEM((176 * 16,), jnp.int32),
            pltpu.VMEM((G,), jnp.int32),
            pltpu.VMEM((G,), jnp.int32),
            pltpu.VMEM((G,), jnp.int32),
            pltpu.VMEM((G,), jnp.int32),
            pltpu.VMEM((2 * G,), jnp.int32),
            pltpu.VMEM((2 * G,), jnp.int32),
            pltpu.VMEM((2 * G,), jnp.int32),
            pltpu.VMEM((2 * G,), jnp.int32),
            pltpu.VMEM((2 * G, D), jnp.float32),
            pltpu.VMEM((2 * G, D), jnp.float32),
            pltpu.VMEM((2 * G, D), jnp.float32),
            pltpu.VMEM((2 * G, D), jnp.float32),
            pltpu.VMEM_SHARED((2 * AGG_ROWS, D), jnp.float32),
            pltpu.SemaphoreType.DMA,
            pltpu.SemaphoreType.DMA,
            pltpu.SemaphoreType.DMA,
            pltpu.SemaphoreType.DMA,
            pltpu.SemaphoreType.DMA,
            pltpu.SemaphoreType.DMA,
            pltpu.SemaphoreType.DMA,
            pltpu.SemaphoreType.DMA,
        ],
    )(_edge_body)
    acc2 = edge_agg(src, dst, h, z128, ones128)

    out = pl.pallas_call(
        _final_body,
        out_shape=jax.ShapeDtypeStruct((S, 1), jnp.float32),
    )(h[:S], acc2, W_self, W_neigh, b_gnn.reshape(1, D),
      W_mlp, b_mlp.reshape(1, 1))
    return jnp.squeeze(out, axis=-1)


# speedup vs baseline: 13.6061x; 1.9678x over previous
import functools

import jax
import jax.numpy as jnp
from jax import lax
from jax.experimental import pallas as pl
from jax.experimental.pallas import tpu as pltpu
from jax.experimental.pallas import tpu_sc as plsc

N = 10000
E = 320000
D = 128
V = 100000
S = 1024

NC = 2
NS = 16
NW = NC * NS

ROWS_PER_W = 320
GCH = 80

EP = 327680
EPW = EP // NW
NG = EPW // 16
G = 64
AGG_ROWS = 1152
RPT = AGG_ROWS // NS
TRASH = S


def _emb_body(nid_hbm, table_hbm, out_hbm, idx_v, rows_v, sem):
    c = lax.axis_index("c")
    s = lax.axis_index("s")
    w = s * NC + c
    base = w * ROWS_PER_W
    for k in range(ROWS_PER_W // GCH):
        b = base + k * GCH

        @pl.when(b < N)
        def _():
            pltpu.sync_copy(nid_hbm.at[pl.ds(b, GCH)], idx_v)
            pltpu.async_copy(table_hbm.at[idx_v], rows_v, sem).wait()
            pltpu.sync_copy(rows_v, out_hbm.at[pl.ds(b, GCH)])


def _edge_body(src_hbm, dst_hbm, gbase_hbm, h_hbm, z128_hbm, ones_hbm,
               acc_out,
               esrc, edst, btab, sel_src, sel_dst,
               ss0, ss1, ss2, ss3, st0, st1, st2, st3,
               rw0, rw1, rw2, rw3, acc_sh,
               g0, g1, g2, g3, sc0, sc1, sc2, sc3):
    c = lax.axis_index("c")
    s = lax.axis_index("s")
    w = s * NC + c

    pltpu.sync_copy(src_hbm.at[pl.ds(w * EPW, EPW)], esrc)
    pltpu.sync_copy(dst_hbm.at[pl.ds(w * EPW, EPW)], edst)
    pltpu.sync_copy(gbase_hbm.at[w], btab.at[pl.ds(0, NG)])
    pltpu.sync_copy(z128_hbm.at[pl.ds(0, 2 * RPT)],
                    acc_sh.at[pl.ds(s * 2 * RPT, 2 * RPT)])
    SS = [ss0, ss1, ss2, ss3]
    ST = [st0, st1, st2, st3]
    RW = [rw0, rw1, rw2, rw3]
    GS = [g0, g1, g2, g3]
    SC = [sc0, sc1, sc2, sc3]
    for b in range(4):
        pltpu.sync_copy(ones_hbm, RW[b].at[pl.ds(G, G)])
    plsc.subcore_barrier()

    lane = jax.lax.iota(jnp.int32, 16)
    shidx = [jnp.maximum(lane - k, 0) for k in (1, 2, 4, 8)]
    shkeep = [jnp.int32(1) - lax.shift_right_logical(lane - k, 31)
              for k in (1, 2, 4, 8)]
    target = lane + jnp.int32(1)
    pos0 = jnp.zeros((16,), jnp.int32)

    def cbody(i, carry):
        for u in range(5):
            off = i * 80 + u * 16
            gidx = i * 5 + u
            base = btab[pl.ds(gidx, 16)][0]
            dv = edst[pl.ds(off, 16)]
            sv = esrc[pl.ds(off, 16)]
            mi = lax.shift_right_logical(dv - jnp.int32(S), 31)
            ips = mi
            for k in range(4):
                sh = ips.at[shidx[k]].get(mode="promise_in_bounds")
                ips = ips + sh * shkeep[k]
            pos = pos0
            for step in (8, 4, 2, 1):
                probe = ips.at[pos + jnp.int32(step - 1)].get(mode="promise_in_bounds")
                ok = lax.shift_right_logical(probe - target, 31)
                pos = pos + ok * jnp.int32(step)
            g = jnp.minimum(pos, jnp.int32(15))
            sel_dst[pl.ds(base, 16)] = dv.at[g].get(mode="promise_in_bounds")
            sel_src[pl.ds(base, 16)] = sv.at[g].get(mode="promise_in_bounds")
        return carry

    lax.fori_loop(0, EPW // 80, cbody, jnp.int32(0))

    cnt = btab[pl.ds(NG - 16, 16)][15]

    for k in range(G // 16):
        sel_dst[pl.ds(cnt + 16 * k, 16)] = jnp.full((16,), TRASH, jnp.int32)
        sel_src[pl.ds(cnt + 16 * k, 16)] = jnp.zeros((16,), jnp.int32)

    nch = (cnt + (G - 1)) >> 6

    def fill_stage(j, b):
        base = j * G
        for k in range(G // 16):
            dvk = sel_dst[pl.ds(base + 16 * k, 16)]
            ST[b][pl.ds(16 * k, 16)] = dvk
            ST[b][pl.ds(G + 16 * k, 16)] = dvk + jnp.int32(AGG_ROWS)
            SS[b][pl.ds(16 * k, 16)] = sel_src[pl.ds(base + 16 * k, 16)]

    def gather_start(b):
        pltpu.async_copy(h_hbm.at[SS[b]], RW[b].at[pl.ds(0, G)], GS[b])

    def gather_wait(b):
        pltpu.make_async_copy(h_hbm.at[SS[b]], RW[b].at[pl.ds(0, G)],
                              GS[b]).wait()

    def scatter_start(b):
        pltpu.async_copy(RW[b], acc_sh.at[ST[b]], SC[b], add=True)

    def scatter_wait(b):
        pltpu.make_async_copy(RW[b], acc_sh.at[ST[b]], SC[b]).wait()

    for k in range(3):
        @pl.when(k < nch)
        def _(k=k):
            fill_stage(jnp.int32(k), k)
            gather_start(k)

    def quad_body(q, carry):
        j0 = q * 4
        for b in range(4):
            j = j0 + b

            @pl.when(j < nch)
            def _(b=b, j=j):
                gather_wait(b)
                scatter_start(b)

                @pl.when(j + 3 < nch)
                def _(b=b, j=j):
                    nb = (b + 3) % 4

                    @pl.when(j >= 1)
                    def _(nb=nb):
                        scatter_wait(nb)

                    fill_stage(j + 3, nb)
                    gather_start(nb)

        return carry

    lax.fori_loop(0, (nch + 3) >> 2, quad_body, jnp.int32(0))

    for b in range(4):
        @pl.when(b < nch)
        def _(b=b):
            scatter_wait(b)

    plsc.subcore_barrier()

    pltpu.sync_copy(acc_sh.at[pl.ds(s * 2 * RPT, 2 * RPT)],
                    acc_out.at[c, pl.ds(s * 2 * RPT, 2 * RPT)])


def _gbase_body(dstt_ref, tri_ref, o_ref):
    m = jnp.where(dstt_ref[...] < S, 1.0, 0.0)
    gs = jnp.sum(m, axis=0, keepdims=True)
    excl = jnp.dot(gs, tri_ref[...], preferred_element_type=jnp.float32)
    o_ref[...] = excl.astype(jnp.int32)


def _enc_body(x_ref, emb_ref, w_ref, b_ref, o_ref):
    o_ref[...] = (
        jnp.dot(x_ref[...], w_ref[...], preferred_element_type=jnp.float32)
        + b_ref[...]
        + emb_ref[...]
    )


def _final_body(h_ref, acc_ref, ws_ref, wn_ref, bg_ref, wm_ref,
                bm_ref, o_ref):
    a = acc_ref[...]
    agg = a[0, :S, :] + a[1, :S, :]
    deg = a[0, AGG_ROWS:AGG_ROWS + S, 0:1] + a[1, AGG_ROWS:AGG_ROWS + S, 0:1]
    mean = agg * (1.0 / jnp.maximum(deg, 1.0))
    h2 = jnp.maximum(
        jnp.dot(h_ref[...], ws_ref[...], preferred_element_type=jnp.float32)
        + jnp.dot(mean, wn_ref[...], preferred_element_type=jnp.float32)
        + bg_ref[...],
        0.0,
    )
    o_ref[...] = jnp.dot(h2, wm_ref[...], preferred_element_type=jnp.float32) + bm_ref[...]


def kernel(x, edge_index, n_id, seed_count, W_enc, b_enc, emb_table,
           W_self, W_neigh, b_gnn, W_mlp, b_mlp):
    del seed_count
    mesh = plsc.VectorSubcoreMesh(core_axis_name="c", subcore_axis_name="s")

    emb_gather = functools.partial(
        pl.kernel,
        out_type=jax.ShapeDtypeStruct((N, D), jnp.float32),
        mesh=mesh,
        scratch_types=[
            pltpu.VMEM((GCH,), jnp.int32),
            pltpu.VMEM((GCH, D), jnp.float32),
            pltpu.SemaphoreType.DMA,
        ],
    )(_emb_body)
    emb_rows = emb_gather(n_id, emb_table)

    R = 2000
    h = pl.pallas_call(
        _enc_body,
        grid=(N // R,),
        in_specs=[
            pl.BlockSpec((R, D), lambda i: (i, 0)),
            pl.BlockSpec((R, D), lambda i: (i, 0)),
            pl.BlockSpec((D, D), lambda i: (0, 0)),
            pl.BlockSpec((1, D), lambda i: (0, 0)),
        ],
        out_specs=pl.BlockSpec((R, D), lambda i: (i, 0)),
        out_shape=jax.ShapeDtypeStruct((N, D), jnp.float32),
    )(x, emb_rows, W_enc, b_enc.reshape(1, D))

    src = jnp.concatenate([edge_index[0], jnp.zeros((EP - E,), jnp.int32)])
    dst = jnp.concatenate([edge_index[1], jnp.full((EP - E,), S, jnp.int32)])
    dst_t = dst.reshape(EP // 16, 16).T
    tri = (jax.lax.broadcasted_iota(jnp.int32, (NG, NG), 0)
           < jax.lax.broadcasted_iota(jnp.int32, (NG, NG), 1)).astype(jnp.float32)
    gbase = pl.pallas_call(
        _gbase_body,
        grid=(NW,),
        in_specs=[
            pl.BlockSpec((16, NG), lambda i: (0, i)),
            pl.BlockSpec((NG, NG), lambda i: (0, 0)),
        ],
        out_specs=pl.BlockSpec((1, NG), lambda i: (0, i)),
        out_shape=jax.ShapeDtypeStruct((1, NW * NG), jnp.int32),
    )(dst_t, tri)
    z128 = jnp.zeros((2 * RPT, D), jnp.float32)
    ones128 = jnp.ones((G, D), jnp.float32)
    edge_agg = functools.partial(
        pl.kernel,
        out_type=jax.ShapeDtypeStruct((NC, 2 * AGG_ROWS, D), jnp.float32),
        mesh=mesh,
        scratch_types=[
            pltpu.VMEM((EPW,), jnp.int32),
            pltpu.VMEM((EPW,), jnp.int32),
            pltpu.VMEM((NG + 16,), jnp.int32),
            pltpu.VMEM((EPW + G + 16,), jnp.int32),
            pltpu.VMEM((EPW + G + 16,), jnp.int32),
            pltpu.VMEM((G,), jnp.int32),
            pltpu.VMEM((G,), jnp.int32),
            pltpu.VMEM((G,), jnp.int32),
            pltpu.VMEM((G,), jnp.int32),
            pltpu.VMEM((2 * G,), jnp.int32),
            pltpu.VMEM((2 * G,), jnp.int32),
            pltpu.VMEM((2 * G,), jnp.int32),
            pltpu.VMEM((2 * G,), jnp.int32),
            pltpu.VMEM((2 * G, D), jnp.float32),
            pltpu.VMEM((2 * G, D), jnp.float32),
            pltpu.VMEM((2 * G, D), jnp.float32),
            pltpu.VMEM((2 * G, D), jnp.float32),
            pltpu.VMEM_SHARED((2 * AGG_ROWS, D), jnp.float32),
            pltpu.SemaphoreType.DMA,
            pltpu.SemaphoreType.DMA,
            pltpu.SemaphoreType.DMA,
            pltpu.SemaphoreType.DMA,
            pltpu.SemaphoreType.DMA,
            pltpu.SemaphoreType.DMA,
            pltpu.SemaphoreType.DMA,
            pltpu.SemaphoreType.DMA,
        ],
    )(_edge_body)
    acc2 = edge_agg(src, dst, gbase.reshape(NW, NG), h, z128, ones128)

    out = pl.pallas_call(
        _final_body,
        out_shape=jax.ShapeDtypeStruct((S, 1), jnp.float32),
    )(h[:S], acc2, W_self, W_neigh, b_gnn.reshape(1, D),
      W_mlp, b_mlp.reshape(1, 1))
    return jnp.squeeze(out, axis=-1)


# speedup vs baseline: 17.1232x; 1.2585x over previous
import functools

import jax
import jax.numpy as jnp
from jax import lax
from jax.experimental import pallas as pl
from jax.experimental.pallas import tpu as pltpu
from jax.experimental.pallas import tpu_sc as plsc

N = 10000
E = 320000
D = 128
V = 100000
S = 1024

NC = 2
NS = 16
NW = NC * NS

ROWS_PER_W = 320
GCH = 80

EPW = E // NW
G = 64
AGG_ROWS = 1152
RPT = AGG_ROWS // NS
TRASH = S


def _emb_body(nid_hbm, table_hbm, out_hbm, idx_v, rows_v, sem):
    c = lax.axis_index("c")
    s = lax.axis_index("s")
    w = s * NC + c
    base = w * ROWS_PER_W
    for k in range(ROWS_PER_W // GCH):
        b = base + k * GCH

        @pl.when(b < N)
        def _():
            pltpu.sync_copy(nid_hbm.at[pl.ds(b, GCH)], idx_v)
            pltpu.async_copy(table_hbm.at[idx_v], rows_v, sem).wait()
            pltpu.sync_copy(rows_v, out_hbm.at[pl.ds(b, GCH)])


def _edge_body(src_hbm, dst_hbm, h_hbm, z128_hbm, ones_hbm,
               acc_out,
               esrc, edst, stage_src0, stage_src1, stage2_0, stage2_1,
               rows2_0, rows2_1, acc_sh, gsem0, gsem1):
    c = lax.axis_index("c")
    s = lax.axis_index("s")
    w = s * NC + c
    sel_src = esrc
    sel_dst = edst

    pltpu.sync_copy(src_hbm.at[pl.ds(w * EPW, EPW)], esrc.at[pl.ds(0, EPW)])
    pltpu.sync_copy(dst_hbm.at[pl.ds(w * EPW, EPW)], edst.at[pl.ds(0, EPW)])
    pltpu.sync_copy(z128_hbm.at[pl.ds(0, 2 * RPT)],
                    acc_sh.at[pl.ds(s * 2 * RPT, 2 * RPT)])
    pltpu.sync_copy(ones_hbm, rows2_0.at[pl.ds(G, G)])
    pltpu.sync_copy(ones_hbm, rows2_1.at[pl.ds(G, G)])
    plsc.subcore_barrier()

    lane = jax.lax.iota(jnp.int32, 16)
    def cbody(i, cnt):
        for u in range(5):
            off = i * 80 + u * 16
            dv = edst[pl.ds(off, 16)]
            sv = esrc[pl.ds(off, 16)]
            mi = lax.shift_right_logical(dv - jnp.int32(S), 31)
            ips = mi
            for k in (1, 2, 4, 8):
                sh = ips.at[jnp.maximum(lane - k, 0)].get(mode="promise_in_bounds")
                keep = jnp.int32(1) - lax.shift_right_logical(lane - k, 31)
                ips = ips + sh * keep
            target = lane + jnp.int32(1)
            pos = jnp.zeros((16,), jnp.int32)
            for step in (8, 4, 2, 1):
                probe = ips.at[pos + jnp.int32(step - 1)].get(mode="promise_in_bounds")
                ok = lax.shift_right_logical(probe - target, 31)
                pos = pos + ok * jnp.int32(step)
            g = jnp.minimum(pos, jnp.int32(15))
            sel_dst[pl.ds(cnt, 16)] = dv.at[g].get(mode="promise_in_bounds")
            sel_src[pl.ds(cnt, 16)] = sv.at[g].get(mode="promise_in_bounds")
            cnt = cnt + ips[15]
        return cnt

    cnt = lax.fori_loop(0, EPW // 80, cbody, jnp.int32(0))

    for k in range(G // 16):
        sel_dst[pl.ds(cnt + 16 * k, 16)] = jnp.full((16,), TRASH, jnp.int32)
        sel_src[pl.ds(cnt + 16 * k, 16)] = jnp.zeros((16,), jnp.int32)

    nch = (cnt + (G - 1)) >> 6

    def fill_stage(j, stage_src_b, stage2_b):
        base = j * G
        for k in range(G // 16):
            dvk = sel_dst[pl.ds(base + 16 * k, 16)]
            stage2_b[pl.ds(16 * k, 16)] = dvk
            stage2_b[pl.ds(G + 16 * k, 16)] = dvk + jnp.int32(AGG_ROWS)
            stage_src_b[pl.ds(16 * k, 16)] = sel_src[pl.ds(base + 16 * k, 16)]

    def gather_start(stage_src_b, rows2_b, sem_b):
        return pltpu.async_copy(h_hbm.at[stage_src_b], rows2_b.at[pl.ds(0, G)],
                                sem_b)

    @pl.when(nch > 0)
    def _():
        fill_stage(jnp.int32(0), stage_src0, stage2_0)
        gather_start(stage_src0, rows2_0, gsem0)

    def pair_body(p, carry):
        j0 = p * 2
        pltpu.make_async_copy(h_hbm.at[stage_src0], rows2_0.at[pl.ds(0, G)],
                              gsem0).wait()

        @pl.when(j0 + 1 < nch)
        def _():
            fill_stage(j0 + 1, stage_src1, stage2_1)
            gather_start(stage_src1, rows2_1, gsem1)

        pltpu.sync_copy(rows2_0, acc_sh.at[stage2_0], add=True)

        @pl.when(j0 + 1 < nch)
        def _():
            pltpu.make_async_copy(h_hbm.at[stage_src1], rows2_1.at[pl.ds(0, G)],
                                  gsem1).wait()

            @pl.when(j0 + 2 < nch)
            def _():
                fill_stage(j0 + 2, stage_src0, stage2_0)
                gather_start(stage_src0, rows2_0, gsem0)

            pltpu.sync_copy(rows2_1, acc_sh.at[stage2_1], add=True)

        return carry

    lax.fori_loop(0, (nch + 1) >> 1, pair_body, jnp.int32(0))
    plsc.subcore_barrier()

    pltpu.sync_copy(acc_sh.at[pl.ds(s * 2 * RPT, 2 * RPT)],
                    acc_out.at[c, pl.ds(s * 2 * RPT, 2 * RPT)])


def _enc_body(x_ref, emb_ref, w_ref, b_ref, o_ref):
    o_ref[...] = (
        jnp.dot(x_ref[...], w_ref[...], preferred_element_type=jnp.float32)
        + b_ref[...]
        + emb_ref[...]
    )


def _final_body(h_ref, acc_ref, ws_ref, wn_ref, bg_ref, wm_ref,
                bm_ref, o_ref):
    a = acc_ref[...]
    agg = a[0, :S, :] + a[1, :S, :]
    deg = a[0, AGG_ROWS:AGG_ROWS + S, 0:1] + a[1, AGG_ROWS:AGG_ROWS + S, 0:1]
    mean = agg * (1.0 / jnp.maximum(deg, 1.0))
    h2 = jnp.maximum(
        jnp.dot(h_ref[...], ws_ref[...], preferred_element_type=jnp.float32)
        + jnp.dot(mean, wn_ref[...], preferred_element_type=jnp.float32)
        + bg_ref[...],
        0.0,
    )
    o_ref[...] = jnp.dot(h2, wm_ref[...], preferred_element_type=jnp.float32) + bm_ref[...]


def kernel(x, edge_index, n_id, seed_count, W_enc, b_enc, emb_table,
           W_self, W_neigh, b_gnn, W_mlp, b_mlp):
    del seed_count
    mesh = plsc.VectorSubcoreMesh(core_axis_name="c", subcore_axis_name="s")

    emb_gather = functools.partial(
        pl.kernel,
        out_type=jax.ShapeDtypeStruct((N, D), jnp.float32),
        mesh=mesh,
        scratch_types=[
            pltpu.VMEM((GCH,), jnp.int32),
            pltpu.VMEM((GCH, D), jnp.float32),
            pltpu.SemaphoreType.DMA,
        ],
    )(_emb_body)
    emb_rows = emb_gather(n_id, emb_table)

    R = 2000
    h = pl.pallas_call(
        _enc_body,
        grid=(N // R,),
        in_specs=[
            pl.BlockSpec((R, D), lambda i: (i, 0)),
            pl.BlockSpec((R, D), lambda i: (i, 0)),
            pl.BlockSpec((D, D), lambda i: (0, 0)),
            pl.BlockSpec((1, D), lambda i: (0, 0)),
        ],
        out_specs=pl.BlockSpec((R, D), lambda i: (i, 0)),
        out_shape=jax.ShapeDtypeStruct((N, D), jnp.float32),
    )(x, emb_rows, W_enc, b_enc.reshape(1, D))

    src = edge_index[0]
    dst = edge_index[1]
    z128 = jnp.zeros((2 * RPT, D), jnp.float32)
    ones128 = jnp.ones((G, D), jnp.float32)
    edge_agg = functools.partial(
        pl.kernel,
        out_type=jax.ShapeDtypeStruct((NC, 2 * AGG_ROWS, D), jnp.float32),
        mesh=mesh,
        scratch_types=[
            pltpu.VMEM((EPW + G + 16,), jnp.int32),
            pltpu.VMEM((EPW + G + 16,), jnp.int32),
            pltpu.VMEM((G,), jnp.int32),
            pltpu.VMEM((G,), jnp.int32),
            pltpu.VMEM((2 * G,), jnp.int32),
            pltpu.VMEM((2 * G,), jnp.int32),
            pltpu.VMEM((2 * G, D), jnp.float32),
            pltpu.VMEM((2 * G, D), jnp.float32),
            pltpu.VMEM_SHARED((2 * AGG_ROWS, D), jnp.float32),
            pltpu.SemaphoreType.DMA,
            pltpu.SemaphoreType.DMA,
        ],
    )(_edge_body)
    acc2 = edge_agg(src, dst, h, z128, ones128)

    out = pl.pallas_call(
        _final_body,
        out_shape=jax.ShapeDtypeStruct((S, 1), jnp.float32),
    )(h[:S], acc2, W_self, W_neigh, b_gnn.reshape(1, D),
      W_mlp, b_mlp.reshape(1, 1))
    return jnp.squeeze(out, axis=-1)
